# Initial kernel scaffold; baseline (speedup 1.0000x reference)
import functools

import jax
import jax.numpy as jnp
from jax import lax
from jax.experimental import pallas as pl
from jax.experimental.pallas import tpu as pltpu
from jax.experimental.pallas import tpu_sc as plsc

N = 10000
E = 320000
IN_DIM = 128
HID = 256
L = 4

NC = 2
NS = 16
K = 80
EPT = E // NS
NWIN = EPT // K
ROWS_PER_SUB = N // NS
HALF = HID // 2


def _ln(x, g, b):
    m = x.mean(-1, keepdims=True)
    v = ((x - m) ** 2).mean(-1, keepdims=True)
    return (x - m) * jax.lax.rsqrt(v + 1e-5) * g + b


def _gelu(x):
    return 0.5 * x * (1.0 + lax.erf(x * (2.0 ** -0.5)))


def _sc_body(with_deg, *refs):
    if with_deg:
        (xp, src_h, dst_h, agg, deg,
         src0, src1, dst0, dst1, rows0, rows1, ones_v, zbuf, acc, dacc,
         semg0, semg1) = refs
    else:
        (xp, src_h, dst_h, agg,
         src0, src1, dst0, dst1, rows0, rows1, zbuf, acc,
         semg0, semg1) = refs

    c = lax.axis_index("c")
    s = lax.axis_index("s")
    ebase = s * EPT
    r0 = s * ROWS_PER_SUB

    zeros16 = jnp.zeros((16,), jnp.float32)

    def zrow(i, carry):
        for j in range(HALF // 16):
            zbuf[i, pl.ds(j * 16, 16)] = zeros16
        return carry

    lax.fori_loop(0, ROWS_PER_SUB, zrow, 0)
    pltpu.sync_copy(zbuf, acc.at[pl.ds(r0, ROWS_PER_SUB)])
    if with_deg:
        def orow(i, carry):
            ones_v[i, pl.ds(0, 16)] = jnp.ones((16,), jnp.float32)
            return carry
        lax.fori_loop(0, K, orow, 0)
        pltpu.sync_copy(zbuf.at[pl.ds(0, ROWS_PER_SUB), pl.ds(0, 16)],
                        dacc.at[pl.ds(r0, ROWS_PER_SUB)])
    plsc.subcore_barrier()

    table = xp.at[c]

    def load_idx(w, sb, db):
        e0 = ebase + w * K
        pltpu.sync_copy(src_h.at[pl.ds(e0, K)], sb)
        pltpu.sync_copy(dst_h.at[pl.ds(e0, K)], db)

    def start_gather(sb, rb, sem):
        pltpu.make_async_copy(table.at[sb], rb, sem).start()

    def wait_gather(sb, rb, sem):
        pltpu.make_async_copy(table.at[sb], rb, sem).wait()

    def scatter(rb, db):
        pltpu.sync_copy(rb, acc.at[db], add=True)
        if with_deg:
            pltpu.sync_copy(ones_v, dacc.at[db], add=True)

    load_idx(0, src0, dst0)
    start_gather(src0, rows0, semg0)

    def step(g, carry):
        w = g * 2
        load_idx(w + 1, src1, dst1)
        start_gather(src1, rows1, semg1)
        wait_gather(src0, rows0, semg0)
        scatter(rows0, dst0)
        load_idx(w + 2, src0, dst0)
        start_gather(src0, rows0, semg0)
        wait_gather(src1, rows1, semg1)
        scatter(rows1, dst1)
        return carry

    lax.fori_loop(0, NWIN // 2 - 1, step, 0)
    load_idx(NWIN - 1, src1, dst1)
    start_gather(src1, rows1, semg1)
    wait_gather(src0, rows0, semg0)
    scatter(rows0, dst0)
    wait_gather(src1, rows1, semg1)
    scatter(rows1, dst1)

    plsc.subcore_barrier()
    pltpu.sync_copy(acc.at[pl.ds(r0, ROWS_PER_SUB)],
                    agg.at[c].at[pl.ds(r0, ROWS_PER_SUB)])
    if with_deg:
        pltpu.sync_copy(dacc.at[pl.ds(r0, ROWS_PER_SUB)],
                        deg.at[c].at[pl.ds(r0, ROWS_PER_SUB)])


def _make_sc(with_deg):
    out_type = [jax.ShapeDtypeStruct((NC, N, HALF), jnp.float32)]
    if with_deg:
        out_type.append(jax.ShapeDtypeStruct((NC, N, 16), jnp.float32))
    scratch = [
        pltpu.VMEM((K,), jnp.int32),
        pltpu.VMEM((K,), jnp.int32),
        pltpu.VMEM((K,), jnp.int32),
        pltpu.VMEM((K,), jnp.int32),
        pltpu.VMEM((K, HALF), jnp.float32),
        pltpu.VMEM((K, HALF), jnp.float32),
    ]
    if with_deg:
        scratch.append(pltpu.VMEM((K, 16), jnp.float32))
    scratch.append(pltpu.VMEM((ROWS_PER_SUB, HALF), jnp.float32))
    scratch.append(pltpu.VMEM_SHARED((N, HALF), jnp.float32))
    if with_deg:
        scratch.append(pltpu.VMEM_SHARED((N, 16), jnp.float32))
    scratch += [pltpu.SemaphoreType.DMA, pltpu.SemaphoreType.DMA]
    mesh = plsc.VectorSubcoreMesh(core_axis_name="c", subcore_axis_name="s")
    return pl.kernel(
        functools.partial(_sc_body, with_deg),
        out_type=tuple(out_type),
        mesh=mesh,
        scratch_types=scratch,
        name=f"sc_segsum{'_deg' if with_deg else ''}",
    )


_sc_segsum_deg = _make_sc(True)
_sc_segsum = _make_sc(False)


R = 1000
G = N // R


def _row(spec_shape):
    nd = len(spec_shape)
    if nd == 2:
        return pl.BlockSpec((R, spec_shape[1]), lambda i: (i, 0))
    return pl.BlockSpec((spec_shape[0], R, spec_shape[2]), lambda i: (0, i, 0))


def _full(shape):
    nd = len(shape)
    return pl.BlockSpec(shape, lambda i: (0,) * nd)


def _in_proj_body(x, w1, b1, w2, b2, lg, lb, blg, blb, wp, bp, wr,
                  xcur_o, xp_o, xr_o):
    h = _gelu(jnp.dot(x[...], w1[...], preferred_element_type=jnp.float32)
              + b1[...])
    h = jnp.dot(h, w2[...], preferred_element_type=jnp.float32) + b2[...]
    h = _ln(h, lg[...], lb[...])
    xcur_o[...] = h
    xln = _ln(h, blg[...], blb[...])
    xp = jnp.maximum(
        jnp.dot(xln, wp[...], preferred_element_type=jnp.float32) + bp[...],
        0.0)
    xp_o[0] = xp[:, :HALF]
    xp_o[1] = xp[:, HALF:]
    xr_o[...] = jnp.dot(xln, wr[...], preferred_element_type=jnp.float32)


def _in_proj(x, W1, b1, W2, b2, lg, lb, blg, blb, wp, bp, wr):
    return pl.pallas_call(
        _in_proj_body,
        grid=(G,),
        in_specs=[
            _row((N, IN_DIM)),
            _full((IN_DIM, 2 * HID)), _full((1, 2 * HID)),
            _full((2 * HID, HID)), _full((1, HID)),
            _full((1, HID)), _full((1, HID)),
            _full((1, HID)), _full((1, HID)),
            _full((HID, HID)), _full((1, HID)),
            _full((HID, HID)),
        ],
        out_specs=[_row((N, HID)), _row((2, N, HALF)), _row((N, HID))],
        out_shape=[
            jax.ShapeDtypeStruct((N, HID), jnp.float32),
            jax.ShapeDtypeStruct((NC, N, HALF), jnp.float32),
            jax.ShapeDtypeStruct((N, HID), jnp.float32),
        ],
        name="tc_in_proj",
    )(x, W1, b1, W2, b2, lg, lb, blg, blb, wp, bp, wr)


def _post_body(last, xcur, xr, agg, deg, wl, bl, wg, bg,
               blg, blb, wp, bp, wr, *outs):
    d = deg[0, :, 0:1]
    mean = jnp.concatenate([agg[0], agg[1]], axis=-1) / jnp.maximum(d, 1.0)
    out = (jnp.dot(mean, wl[...], preferred_element_type=jnp.float32)
           + bl[...] + xr[...])
    nrm = jnp.sqrt(jnp.sum(out * out, axis=-1, keepdims=True))
    out = out / jnp.maximum(nrm, 1e-12)
    hi = xcur[...] + _gelu(out)
    xc = xcur[...]
    gate_in = jnp.concatenate([xc, hi], axis=-1)
    g = jax.nn.sigmoid(
        jnp.dot(gate_in, wg[...], preferred_element_type=jnp.float32)
        + bg[...])
    xnew = xc + g * hi
    if last:
        outs[0][...] = _ln(xnew, blg[...], blb[...])
    else:
        xcur_o, xp_o, xr_o = outs
        xcur_o[...] = xnew
        xln = _ln(xnew, blg[...], blb[...])
        xp = jnp.maximum(
            jnp.dot(xln, wp[...], preferred_element_type=jnp.float32)
            + bp[...], 0.0)
        xp_o[0] = xp[:, :HALF]
        xp_o[1] = xp[:, HALF:]
        xr_o[...] = jnp.dot(xln, wr[...], preferred_element_type=jnp.float32)


def _post(last, xcur, xr, agg, deg, wl, bl, wg, bg, blg, blb, wp, bp, wr):
    if last:
        out_specs = [_row((N, HID))]
        out_shape = [jax.ShapeDtypeStruct((N, HID), jnp.float32)]
    else:
        out_specs = [_row((N, HID)), _row((2, N, HALF)), _row((N, HID))]
        out_shape = [
            jax.ShapeDtypeStruct((N, HID), jnp.float32),
            jax.ShapeDtypeStruct((NC, N, HALF), jnp.float32),
            jax.ShapeDtypeStruct((N, HID), jnp.float32),
        ]
    res = pl.pallas_call(
        functools.partial(_post_body, last),
        grid=(G,),
        in_specs=[
            _row((N, HID)), _row((N, HID)),
            _row((2, N, HALF)),
            pl.BlockSpec((2, R, 16), lambda i: (0, i, 0)),
            _full((HID, HID)), _full((1, HID)),
            _full((2 * HID, HID)), _full((1, HID)),
            _full((1, HID)), _full((1, HID)),
            _full((HID, HID)), _full((1, HID)),
            _full((HID, HID)),
        ],
        out_specs=out_specs,
        out_shape=out_shape,
        name="tc_post_last" if last else "tc_post",
    )(xcur, xr, agg, deg, wl, bl, wg, bg, blg, blb, wp, bp, wr)
    return res


def kernel(x, edge_index, W1, b1, W2, b2, ln_in_g, ln_in_b, Wp, bp, Wl, bl,
           Wr, blk_ln_g, blk_ln_b, Wg, bg, fn_g, fn_b):
    src = edge_index[0]
    dst = edge_index[1]
    r2 = lambda a: a.reshape(1, -1)

    xcur, xp, xr = _in_proj(
        x, W1, r2(b1), W2, r2(b2), r2(ln_in_g), r2(ln_in_b),
        r2(blk_ln_g[0]), r2(blk_ln_b[0]), Wp[0], r2(bp[0]), Wr[0])

    deg = None
    for i in range(L):
        if i == 0:
            agg, deg = _sc_segsum_deg(xp, src, dst)
        else:
            (agg,) = _sc_segsum(xp, src, dst)
        last = i == L - 1
        if last:
            nblg, nblb = fn_g, fn_b
            nwp, nbp, nwr = Wp[0], bp[0], Wr[0]
        else:
            nblg, nblb = blk_ln_g[i + 1], blk_ln_b[i + 1]
            nwp, nbp, nwr = Wp[i + 1], bp[i + 1], Wr[i + 1]
        res = _post(last, xcur, xr, agg, deg, Wl[i], r2(bl[i]), Wg[i],
                    r2(bg[i]), r2(nblg), r2(nblb), nwp, r2(nbp), nwr)
        if last:
            return res[0]
        xcur, xp, xr = res


# speedup vs baseline: 5.5340x; 5.5340x over previous
import functools

import jax
import jax.numpy as jnp
from jax import lax
from jax.experimental import pallas as pl
from jax.experimental.pallas import tpu as pltpu
from jax.experimental.pallas import tpu_sc as plsc

N = 10000
E = 320000
IN_DIM = 128
HID = 256
L = 4

NC = 2
NS = 16
K = 80
EPT = E // NS
NWIN = EPT // K
NPAD = 10240
ROWS_PER_SUB = NPAD // NS
HALF = HID // 2
ZCH = 64


def _ln(x, g, b):
    m = x.mean(-1, keepdims=True)
    v = ((x - m) ** 2).mean(-1, keepdims=True)
    return (x - m) * jax.lax.rsqrt(v + 1e-5) * g + b


def _gelu(x):
    return 0.5 * x * (1.0 + lax.erf(x * (2.0 ** -0.5)))


def _sc_body(with_deg, *refs):
    if with_deg:
        (xp, src_h, dst_h, agg, deg,
         src0, src1, dst0, dst1, rows0, rows1, ones_v, zbuf, zdbuf, acc,
         dacc, semg0, semg1) = refs
    else:
        (xp, src_h, dst_h, agg,
         src0, src1, dst0, dst1, rows0, rows1, zbuf, acc,
         semg0, semg1) = refs

    c = lax.axis_index("c")
    s = lax.axis_index("s")
    ebase = s * EPT
    r0 = s * ROWS_PER_SUB

    zeros16 = jnp.zeros((16,), jnp.float32)

    def zrow(i, carry):
        for j in range(HALF // 16):
            zbuf[i, pl.ds(j * 16, 16)] = zeros16
        return carry

    lax.fori_loop(0, ZCH, zrow, 0)

    def zcopy(i, carry):
        pltpu.sync_copy(zbuf, acc.at[pl.ds(r0 + i * ZCH, ZCH)])
        return carry

    lax.fori_loop(0, ROWS_PER_SUB // ZCH, zcopy, 0)
    if with_deg:
        def orow(i, carry):
            ones_v[i, pl.ds(0, 16)] = jnp.ones((16,), jnp.float32)
            zdbuf[i, pl.ds(0, 16)] = zeros16
            return carry
        lax.fori_loop(0, K, orow, 0)

        def zdcopy(i, carry):
            pltpu.sync_copy(zdbuf, dacc.at[pl.ds(r0 + i * K, K)])
            return carry
        lax.fori_loop(0, ROWS_PER_SUB // K, zdcopy, 0)
    plsc.subcore_barrier()

    table = xp.at[c]

    def load_idx(w, sb, db):
        e0 = ebase + w * K
        pltpu.sync_copy(src_h.at[pl.ds(e0, K)], sb)
        pltpu.sync_copy(dst_h.at[pl.ds(e0, K)], db)

    def start_gather(sb, rb, sem):
        pltpu.make_async_copy(table.at[sb], rb, sem).start()

    def wait_gather(sb, rb, sem):
        pltpu.make_async_copy(table.at[sb], rb, sem).wait()

    def scatter(rb, db):
        pltpu.sync_copy(rb, acc.at[db], add=True)
        if with_deg:
            pltpu.sync_copy(ones_v, dacc.at[db], add=True)

    load_idx(0, src0, dst0)
    start_gather(src0, rows0, semg0)

    def step(g, carry):
        w = g * 2
        load_idx(w + 1, src1, dst1)
        start_gather(src1, rows1, semg1)
        wait_gather(src0, rows0, semg0)
        scatter(rows0, dst0)
        load_idx(w + 2, src0, dst0)
        start_gather(src0, rows0, semg0)
        wait_gather(src1, rows1, semg1)
        scatter(rows1, dst1)
        return carry

    lax.fori_loop(0, NWIN // 2 - 1, step, 0)
    load_idx(NWIN - 1, src1, dst1)
    start_gather(src1, rows1, semg1)
    wait_gather(src0, rows0, semg0)
    scatter(rows0, dst0)
    wait_gather(src1, rows1, semg1)
    scatter(rows1, dst1)

    plsc.subcore_barrier()
    pltpu.sync_copy(acc.at[pl.ds(r0, ROWS_PER_SUB)],
                    agg.at[c].at[pl.ds(r0, ROWS_PER_SUB)])
    if with_deg:
        pltpu.sync_copy(dacc.at[pl.ds(r0, ROWS_PER_SUB)],
                        deg.at[c].at[pl.ds(r0, ROWS_PER_SUB)])


def _make_sc(with_deg):
    out_type = [jax.ShapeDtypeStruct((NC, NPAD, HALF), jnp.float32)]
    if with_deg:
        out_type.append(jax.ShapeDtypeStruct((NC, NPAD, 16), jnp.float32))
    scratch = [
        pltpu.VMEM((K,), jnp.int32),
        pltpu.VMEM((K,), jnp.int32),
        pltpu.VMEM((K,), jnp.int32),
        pltpu.VMEM((K,), jnp.int32),
        pltpu.VMEM((K, HALF), jnp.float32),
        pltpu.VMEM((K, HALF), jnp.float32),
    ]
    if with_deg:
        scratch.append(pltpu.VMEM((K, 16), jnp.float32))
    scratch.append(pltpu.VMEM((ZCH, HALF), jnp.float32))
    if with_deg:
        scratch.append(pltpu.VMEM((K, 16), jnp.float32))
    scratch.append(pltpu.VMEM_SHARED((NPAD, HALF), jnp.float32))
    if with_deg:
        scratch.append(pltpu.VMEM_SHARED((NPAD, 16), jnp.float32))
    scratch += [pltpu.SemaphoreType.DMA, pltpu.SemaphoreType.DMA]
    mesh = plsc.VectorSubcoreMesh(core_axis_name="c", subcore_axis_name="s",
                                  num_cores=NC, num_subcores=NS)
    return pl.kernel(
        functools.partial(_sc_body, with_deg),
        out_type=tuple(out_type),
        mesh=mesh,
        scratch_types=scratch,
        compiler_params=pltpu.CompilerParams(use_tc_tiling_on_sc=False),
        name=f"sc_segsum{'_deg' if with_deg else ''}",
    )


_sc_cache = {}


def _sc_segsum_deg(xp, src, dst):
    if True not in _sc_cache:
        _sc_cache[True] = _make_sc(True)
    return _sc_cache[True](xp, src, dst)


def _sc_segsum(xp, src, dst):
    if False not in _sc_cache:
        _sc_cache[False] = _make_sc(False)
    return _sc_cache[False](xp, src, dst)


R = 1000
G = N // R


def _row(spec_shape):
    nd = len(spec_shape)
    if nd == 2:
        return pl.BlockSpec((R, spec_shape[1]), lambda i: (i, 0))
    return pl.BlockSpec((spec_shape[0], R, spec_shape[2]), lambda i: (0, i, 0))


def _full(shape):
    nd = len(shape)
    return pl.BlockSpec(shape, lambda i: (0,) * nd)


def _in_proj_body(x, w1, b1, w2, b2, lg, lb, blg, blb, wp, bp, wr,
                  xcur_o, xp_o, xr_o):
    h = _gelu(jnp.dot(x[...], w1[...], preferred_element_type=jnp.float32)
              + b1[...])
    h = jnp.dot(h, w2[...], preferred_element_type=jnp.float32) + b2[...]
    h = _ln(h, lg[...], lb[...])
    xcur_o[...] = h
    xln = _ln(h, blg[...], blb[...])
    xp = jnp.maximum(
        jnp.dot(xln, wp[...], preferred_element_type=jnp.float32) + bp[...],
        0.0)
    xp_o[0] = xp[:, :HALF]
    xp_o[1] = xp[:, HALF:]
    xr_o[...] = jnp.dot(xln, wr[...], preferred_element_type=jnp.float32)


def _in_proj(x, W1, b1, W2, b2, lg, lb, blg, blb, wp, bp, wr):
    return pl.pallas_call(
        _in_proj_body,
        grid=(G,),
        in_specs=[
            _row((N, IN_DIM)),
            _full((IN_DIM, 2 * HID)), _full((1, 2 * HID)),
            _full((2 * HID, HID)), _full((1, HID)),
            _full((1, HID)), _full((1, HID)),
            _full((1, HID)), _full((1, HID)),
            _full((HID, HID)), _full((1, HID)),
            _full((HID, HID)),
        ],
        out_specs=[_row((N, HID)), _row((2, N, HALF)), _row((N, HID))],
        out_shape=[
            jax.ShapeDtypeStruct((N, HID), jnp.float32),
            jax.ShapeDtypeStruct((NC, N, HALF), jnp.float32),
            jax.ShapeDtypeStruct((N, HID), jnp.float32),
        ],
        name="tc_in_proj",
    )(x, W1, b1, W2, b2, lg, lb, blg, blb, wp, bp, wr)


def _post_body(last, xcur, xr, agg, deg, wl, bl, wg, bg,
               blg, blb, wp, bp, wr, *outs):
    d = deg[0, :, 0:1]
    mean = jnp.concatenate([agg[0], agg[1]], axis=-1) / jnp.maximum(d, 1.0)
    out = (jnp.dot(mean, wl[...], preferred_element_type=jnp.float32)
           + bl[...] + xr[...])
    nrm = jnp.sqrt(jnp.sum(out * out, axis=-1, keepdims=True))
    out = out / jnp.maximum(nrm, 1e-12)
    hi = xcur[...] + _gelu(out)
    xc = xcur[...]
    gate_in = jnp.concatenate([xc, hi], axis=-1)
    g = jax.nn.sigmoid(
        jnp.dot(gate_in, wg[...], preferred_element_type=jnp.float32)
        + bg[...])
    xnew = xc + g * hi
    if last:
        outs[0][...] = _ln(xnew, blg[...], blb[...])
    else:
        xcur_o, xp_o, xr_o = outs
        xcur_o[...] = xnew
        xln = _ln(xnew, blg[...], blb[...])
        xp = jnp.maximum(
            jnp.dot(xln, wp[...], preferred_element_type=jnp.float32)
            + bp[...], 0.0)
        xp_o[0] = xp[:, :HALF]
        xp_o[1] = xp[:, HALF:]
        xr_o[...] = jnp.dot(xln, wr[...], preferred_element_type=jnp.float32)


def _post(last, xcur, xr, agg, deg, wl, bl, wg, bg, blg, blb, wp, bp, wr):
    if last:
        out_specs = [_row((N, HID))]
        out_shape = [jax.ShapeDtypeStruct((N, HID), jnp.float32)]
    else:
        out_specs = [_row((N, HID)), _row((2, N, HALF)), _row((N, HID))]
        out_shape = [
            jax.ShapeDtypeStruct((N, HID), jnp.float32),
            jax.ShapeDtypeStruct((NC, N, HALF), jnp.float32),
            jax.ShapeDtypeStruct((N, HID), jnp.float32),
        ]
    res = pl.pallas_call(
        functools.partial(_post_body, last),
        grid=(G,),
        in_specs=[
            _row((N, HID)), _row((N, HID)),
            _row((2, N, HALF)),
            pl.BlockSpec((2, R, 16), lambda i: (0, i, 0)),
            _full((HID, HID)), _full((1, HID)),
            _full((2 * HID, HID)), _full((1, HID)),
            _full((1, HID)), _full((1, HID)),
            _full((HID, HID)), _full((1, HID)),
            _full((HID, HID)),
        ],
        out_specs=out_specs,
        out_shape=out_shape,
        name="tc_post_last" if last else "tc_post",
    )(xcur, xr, agg, deg, wl, bl, wg, bg, blg, blb, wp, bp, wr)
    return res


def kernel(x, edge_index, W1, b1, W2, b2, ln_in_g, ln_in_b, Wp, bp, Wl, bl,
           Wr, blk_ln_g, blk_ln_b, Wg, bg, fn_g, fn_b):
    src = edge_index[0]
    dst = edge_index[1]
    r2 = lambda a: a.reshape(1, -1)

    xcur, xp, xr = _in_proj(
        x, W1, r2(b1), W2, r2(b2), r2(ln_in_g), r2(ln_in_b),
        r2(blk_ln_g[0]), r2(blk_ln_b[0]), Wp[0], r2(bp[0]), Wr[0])

    deg = None
    for i in range(L):
        if i == 0:
            agg, deg = _sc_segsum_deg(xp, src, dst)
        else:
            (agg,) = _sc_segsum(xp, src, dst)
        last = i == L - 1
        if last:
            nblg, nblb = fn_g, fn_b
            nwp, nbp, nwr = Wp[0], bp[0], Wr[0]
        else:
            nblg, nblb = blk_ln_g[i + 1], blk_ln_b[i + 1]
            nwp, nbp, nwr = Wp[i + 1], bp[i + 1], Wr[i + 1]
        res = _post(last, xcur, xr, agg, deg, Wl[i], r2(bl[i]), Wg[i],
                    r2(bg[i]), r2(nblg), r2(nblb), nwp, r2(nbp), nwr)
        if last:
            return res[0]
        xcur, xp, xr = res


# speedup vs baseline: 8.5214x; 1.5398x over previous
import functools

import jax
import jax.numpy as jnp
from jax import lax
from jax.experimental import pallas as pl
from jax.experimental.pallas import tpu as pltpu
from jax.experimental.pallas import tpu_sc as plsc

N = 10000
E = 320000
IN_DIM = 128
HID = 256
L = 4

NC = 2
NS = 16
K = 128
WPT = 160
EPAD = NS * WPT * K
NPAD = 10240
ROWS_PER_SUB = NPAD // NS
HALF = HID // 2
ZCH = 16
NB = 2
CH = 20
NCH = WPT // CH


def _ln(x, g, b):
    m = x.mean(-1, keepdims=True)
    v = ((x - m) ** 2).mean(-1, keepdims=True)
    return (x - m) * jax.lax.rsqrt(v + 1e-5) * g + b


def _gelu(x):
    return 0.5 * x * (1.0 + lax.erf(x * (2.0 ** -0.5)))


def _zero_shared(zb, dest, r0, nrows, width):
    zeros16 = jnp.zeros((16,), jnp.float32)
    zrows = zb.shape[0]

    def zrow(i, carry):
        for j in range(width // 16):
            zb[i, pl.ds(j * 16, 16)] = zeros16
        return carry

    lax.fori_loop(0, zrows, zrow, 0)

    def zcopy(i, carry):
        pltpu.sync_copy(zb, dest.at[pl.ds(r0 + i * zrows, zrows)])
        return carry

    lax.fori_loop(0, nrows // zrows, zcopy, 0)


def _sc_body(xp, src_h, dst_h, agg, src_c, dst_c, rows0, rows1, zbuf, acc,
             sg0, sg1):
    rows = (rows0, rows1)
    sg = (sg0, sg1)

    c = lax.axis_index("c")
    s = lax.axis_index("s")
    r0 = s * ROWS_PER_SUB

    _zero_shared(zbuf, acc, r0, ROWS_PER_SUB, HALF)
    plsc.subcore_barrier()

    table = xp.at[c]

    def start_gather(w, b):
        pltpu.make_async_copy(table.at[src_c.at[w]], rows[b], sg[b]).start()

    def wait_gather(w, b):
        pltpu.make_async_copy(table.at[src_c.at[w]], rows[b], sg[b]).wait()

    def scatter(w, b):
        pltpu.sync_copy(rows[b], acc.at[dst_c.at[w]], add=True)

    def chunk(c2, carry):
        pltpu.sync_copy(src_h.at[s].at[pl.ds(c2 * CH, CH)], src_c)
        pltpu.sync_copy(dst_h.at[s].at[pl.ds(c2 * CH, CH)], dst_c)
        for b in range(NB):
            start_gather(b, b)

        def step(g, carry2):
            for b in range(NB):
                w = g * NB + b
                wait_gather(w, b)
                scatter(w, b)
                start_gather(w + NB, b)
            return carry2

        lax.fori_loop(0, CH // NB - 1, step, 0)
        for b in range(NB):
            w = CH - NB + b
            wait_gather(w, b)
            scatter(w, b)
        return carry

    lax.fori_loop(0, NCH, chunk, 0)

    plsc.subcore_barrier()
    pltpu.sync_copy(acc.at[pl.ds(r0, ROWS_PER_SUB)],
                    agg.at[c].at[pl.ds(r0, ROWS_PER_SUB)])


WPD = EPAD // (NC * NS) // K


def _sc_deg_body(dst_h, deg, dst_c, ones_v, zdbuf, dacc):
    c = lax.axis_index("c")
    s = lax.axis_index("s")
    wid = s * NC + c
    r0 = s * ROWS_PER_SUB

    _zero_shared(zdbuf, dacc, r0, ROWS_PER_SUB, 16)

    def orow(i, carry):
        ones_v[i, pl.ds(0, 16)] = jnp.ones((16,), jnp.float32)
        return carry

    lax.fori_loop(0, K, orow, 0)
    plsc.subcore_barrier()

    def chunk(c2, carry):
        pltpu.sync_copy(dst_h.at[wid].at[pl.ds(c2 * CH, CH)], dst_c)

        def step(w, carry2):
            pltpu.sync_copy(ones_v, dacc.at[dst_c.at[w]], add=True)
            return carry2

        lax.fori_loop(0, CH, step, 0)
        return carry

    lax.fori_loop(0, WPD // CH, chunk, 0)

    plsc.subcore_barrier()
    pltpu.sync_copy(dacc.at[pl.ds(r0, ROWS_PER_SUB)],
                    deg.at[c].at[pl.ds(r0, ROWS_PER_SUB)])


def _make_sc():
    scratch = [
        pltpu.VMEM((CH, K), jnp.int32),
        pltpu.VMEM((CH, K), jnp.int32),
        pltpu.VMEM((K, HALF), jnp.float32),
        pltpu.VMEM((K, HALF), jnp.float32),
        pltpu.VMEM((ZCH, HALF), jnp.float32),
        pltpu.VMEM_SHARED((NPAD, HALF), jnp.float32),
        pltpu.SemaphoreType.DMA, pltpu.SemaphoreType.DMA,
    ]
    mesh = plsc.VectorSubcoreMesh(core_axis_name="c", subcore_axis_name="s",
                                  num_cores=NC, num_subcores=NS)
    return pl.kernel(
        _sc_body,
        out_type=jax.ShapeDtypeStruct((NC, NPAD, HALF), jnp.float32),
        mesh=mesh,
        scratch_types=scratch,
        compiler_params=pltpu.CompilerParams(use_tc_tiling_on_sc=False),
        name="sc_segsum",
    )


def _make_sc_deg():
    scratch = [
        pltpu.VMEM((CH, K), jnp.int32),
        pltpu.VMEM((K, 16), jnp.float32),
        pltpu.VMEM((K, 16), jnp.float32),
        pltpu.VMEM_SHARED((NPAD, 16), jnp.float32),
    ]
    mesh = plsc.VectorSubcoreMesh(core_axis_name="c", subcore_axis_name="s",
                                  num_cores=NC, num_subcores=NS)
    return pl.kernel(
        _sc_deg_body,
        out_type=jax.ShapeDtypeStruct((NC, NPAD, 16), jnp.float32),
        mesh=mesh,
        scratch_types=scratch,
        compiler_params=pltpu.CompilerParams(use_tc_tiling_on_sc=False),
        name="sc_deg",
    )


_sc_cache = {}


def _sc_segsum(xp, src, dst):
    if "agg" not in _sc_cache:
        _sc_cache["agg"] = _make_sc()
    return _sc_cache["agg"](xp, src, dst)


def _sc_degcount(dst):
    if "deg" not in _sc_cache:
        _sc_cache["deg"] = _make_sc_deg()
    return _sc_cache["deg"](dst)


R = 1000
G = N // R


def _row(spec_shape):
    nd = len(spec_shape)
    if nd == 2:
        return pl.BlockSpec((R, spec_shape[1]), lambda i: (i, 0))
    return pl.BlockSpec((spec_shape[0], R, spec_shape[2]), lambda i: (0, i, 0))


def _full(shape):
    nd = len(shape)
    return pl.BlockSpec(shape, lambda i: (0,) * nd)


def _in_proj_body(x, w1, b1, w2, b2, lg, lb, blg, blb, wp, bp, wr,
                  xcur_o, xp_o, xr_o):
    h = _gelu(jnp.dot(x[...], w1[...], preferred_element_type=jnp.float32)
              + b1[...])
    h = jnp.dot(h, w2[...], preferred_element_type=jnp.float32) + b2[...]
    h = _ln(h, lg[...], lb[...])
    xcur_o[...] = h
    xln = _ln(h, blg[...], blb[...])
    xp = jnp.maximum(
        jnp.dot(xln, wp[...], preferred_element_type=jnp.float32) + bp[...],
        0.0)
    xp_o[0] = xp[:, :HALF]
    xp_o[1] = xp[:, HALF:]
    xr_o[...] = jnp.dot(xln, wr[...], preferred_element_type=jnp.float32)


def _in_proj(x, W1, b1, W2, b2, lg, lb, blg, blb, wp, bp, wr):
    return pl.pallas_call(
        _in_proj_body,
        grid=(G,),
        in_specs=[
            _row((N, IN_DIM)),
            _full((IN_DIM, 2 * HID)), _full((1, 2 * HID)),
            _full((2 * HID, HID)), _full((1, HID)),
            _full((1, HID)), _full((1, HID)),
            _full((1, HID)), _full((1, HID)),
            _full((HID, HID)), _full((1, HID)),
            _full((HID, HID)),
        ],
        out_specs=[_row((N, HID)), _row((2, N, HALF)), _row((N, HID))],
        out_shape=[
            jax.ShapeDtypeStruct((N, HID), jnp.float32),
            jax.ShapeDtypeStruct((NC, N, HALF), jnp.float32),
            jax.ShapeDtypeStruct((N, HID), jnp.float32),
        ],
        name="tc_in_proj",
    )(x, W1, b1, W2, b2, lg, lb, blg, blb, wp, bp, wr)


def _post_body(last, xcur, xr, agg, deg, wl, bl, wg, bg,
               blg, blb, wp, bp, wr, *outs):
    d = deg[0, :, 0:1] + deg[1, :, 0:1]
    mean = jnp.concatenate([agg[0], agg[1]], axis=-1) / jnp.maximum(d, 1.0)
    out = (jnp.dot(mean, wl[...], preferred_element_type=jnp.float32)
           + bl[...] + xr[...])
    nrm = jnp.sqrt(jnp.sum(out * out, axis=-1, keepdims=True))
    out = out / jnp.maximum(nrm, 1e-12)
    hi = xcur[...] + _gelu(out)
    xc = xcur[...]
    gate_in = jnp.concatenate([xc, hi], axis=-1)
    g = jax.nn.sigmoid(
        jnp.dot(gate_in, wg[...], preferred_element_type=jnp.float32)
        + bg[...])
    xnew = xc + g * hi
    if last:
        outs[0][...] = _ln(xnew, blg[...], blb[...])
    else:
        xcur_o, xp_o, xr_o = outs
        xcur_o[...] = xnew
        xln = _ln(xnew, blg[...], blb[...])
        xp = jnp.maximum(
            jnp.dot(xln, wp[...], preferred_element_type=jnp.float32)
            + bp[...], 0.0)
        xp_o[0] = xp[:, :HALF]
        xp_o[1] = xp[:, HALF:]
        xr_o[...] = jnp.dot(xln, wr[...], preferred_element_type=jnp.float32)


def _post(last, xcur, xr, agg, deg, wl, bl, wg, bg, blg, blb, wp, bp, wr):
    if last:
        out_specs = [_row((N, HID))]
        out_shape = [jax.ShapeDtypeStruct((N, HID), jnp.float32)]
    else:
        out_specs = [_row((N, HID)), _row((2, N, HALF)), _row((N, HID))]
        out_shape = [
            jax.ShapeDtypeStruct((N, HID), jnp.float32),
            jax.ShapeDtypeStruct((NC, N, HALF), jnp.float32),
            jax.ShapeDtypeStruct((N, HID), jnp.float32),
        ]
    res = pl.pallas_call(
        functools.partial(_post_body, last),
        grid=(G,),
        in_specs=[
            _row((N, HID)), _row((N, HID)),
            _row((2, N, HALF)),
            pl.BlockSpec((2, R, 16), lambda i: (0, i, 0)),
            _full((HID, HID)), _full((1, HID)),
            _full((2 * HID, HID)), _full((1, HID)),
            _full((1, HID)), _full((1, HID)),
            _full((HID, HID)), _full((1, HID)),
            _full((HID, HID)),
        ],
        out_specs=out_specs,
        out_shape=out_shape,
        name="tc_post_last" if last else "tc_post",
    )(xcur, xr, agg, deg, wl, bl, wg, bg, blg, blb, wp, bp, wr)
    return res


def kernel(x, edge_index, W1, b1, W2, b2, ln_in_g, ln_in_b, Wp, bp, Wl, bl,
           Wr, blk_ln_g, blk_ln_b, Wg, bg, fn_g, fn_b):
    npad_e = EPAD - E
    fill = jnp.arange(npad_e, dtype=jnp.int32)
    src = jnp.concatenate([edge_index[0], fill % N]).reshape(NS, WPT, K)
    dst = jnp.concatenate([edge_index[1], N + fill % (NPAD - N)]
                          ).reshape(NS, WPT, K)
    r2 = lambda a: a.reshape(1, -1)

    xcur, xp, xr = _in_proj(
        x, W1, r2(b1), W2, r2(b2), r2(ln_in_g), r2(ln_in_b),
        r2(blk_ln_g[0]), r2(blk_ln_b[0]), Wp[0], r2(bp[0]), Wr[0])

    deg = _sc_degcount(dst.reshape(NC * NS, WPD, K))
    for i in range(L):
        agg = _sc_segsum(xp, src, dst)
        last = i == L - 1
        if last:
            nblg, nblb = fn_g, fn_b
            nwp, nbp, nwr = Wp[0], bp[0], Wr[0]
        else:
            nblg, nblb = blk_ln_g[i + 1], blk_ln_b[i + 1]
            nwp, nbp, nwr = Wp[i + 1], bp[i + 1], Wr[i + 1]
        res = _post(last, xcur, xr, agg, deg, Wl[i], r2(bl[i]), Wg[i],
                    r2(bg[i]), r2(nblg), r2(nblb), nwp, r2(nbp), nwr)
        if last:
            return res[0]
        xcur, xp, xr = res


# speedup vs baseline: 9.6405x; 1.1313x over previous
import functools

import jax
import jax.numpy as jnp
from jax import lax
from jax.experimental import pallas as pl
from jax.experimental.pallas import tpu as pltpu
from jax.experimental.pallas import tpu_sc as plsc

N = 10000
E = 320000
IN_DIM = 128
HID = 256
L = 4

NC = 2
NS = 16
K = 96
WPT = 210
EPAD = NS * WPT * K
NPAD = 10240
ROWS_PER_SUB = NPAD // NS
HALF = HID // 2
ZCH = 16
NB = 2
NBUF = 3
CH = 30
NCH = WPT // CH
CHD = 21


def _ln(x, g, b):
    m = x.mean(-1, keepdims=True)
    v = ((x - m) ** 2).mean(-1, keepdims=True)
    return (x - m) * jax.lax.rsqrt(v + 1e-5) * g + b


def _gelu(x):
    return 0.5 * x * (1.0 + lax.erf(x * (2.0 ** -0.5)))


def _zero_shared(zb, dest, r0, nrows, width):
    zeros16 = jnp.zeros((16,), jnp.float32)
    zrows = zb.shape[0]

    def zrow(i, carry):
        for j in range(width // 16):
            zb[i, pl.ds(j * 16, 16)] = zeros16
        return carry

    lax.fori_loop(0, zrows, zrow, 0)

    def zcopy(i, carry):
        pltpu.sync_copy(zb, dest.at[pl.ds(r0 + i * zrows, zrows)])
        return carry

    lax.fori_loop(0, nrows // zrows, zcopy, 0)


def _sc_body(xp, src_h, dst_h, agg, src_c, dst_c, rows0, rows1, rows2, zbuf,
             acc, sg0, sg1, sg2):
    rows = (rows0, rows1, rows2)
    sg = (sg0, sg1, sg2)

    c = lax.axis_index("c")
    s = lax.axis_index("s")
    r0 = s * ROWS_PER_SUB

    _zero_shared(zbuf, acc, r0, ROWS_PER_SUB, HALF)
    plsc.subcore_barrier()

    table = xp.at[c]

    def start_gather(w, b):
        pltpu.make_async_copy(table.at[src_c.at[w]], rows[b], sg[b]).start()

    def wait_gather(w, b):
        pltpu.make_async_copy(table.at[src_c.at[w]], rows[b], sg[b]).wait()

    def scatter(w, b):
        pltpu.sync_copy(rows[b], acc.at[dst_c.at[w]], add=True)

    def chunk(c2, carry):
        pltpu.sync_copy(src_h.at[s].at[pl.ds(c2 * CH, CH)], src_c)
        pltpu.sync_copy(dst_h.at[s].at[pl.ds(c2 * CH, CH)], dst_c)
        for b in range(NB):
            start_gather(b, b)

        def step(g, carry2):
            for j in range(NBUF):
                w = g * NBUF + j
                wait_gather(w, j)
                start_gather(w + NB, (j + NB) % NBUF)
                scatter(w, j)
            return carry2

        lax.fori_loop(0, (CH - NBUF) // NBUF, step, 0)
        for w in range(CH - NBUF, CH):
            b = w % NBUF
            wait_gather(w, b)
            if w + NB < CH:
                start_gather(w + NB, (w + NB) % NBUF)
            scatter(w, b)
        return carry

    lax.fori_loop(0, NCH, chunk, 0)

    plsc.subcore_barrier()
    pltpu.sync_copy(acc.at[pl.ds(r0, ROWS_PER_SUB)],
                    agg.at[c].at[pl.ds(r0, ROWS_PER_SUB)])


WPD = EPAD // (NC * NS) // K


def _sc_deg_body(dst_h, deg, dst_c, ones_v, zdbuf, dacc):
    c = lax.axis_index("c")
    s = lax.axis_index("s")
    wid = s * NC + c
    r0 = s * ROWS_PER_SUB

    _zero_shared(zdbuf, dacc, r0, ROWS_PER_SUB, 16)

    def orow(i, carry):
        ones_v[i, pl.ds(0, 16)] = jnp.ones((16,), jnp.float32)
        return carry

    lax.fori_loop(0, K, orow, 0)
    plsc.subcore_barrier()

    def chunk(c2, carry):
        pltpu.sync_copy(dst_h.at[wid].at[pl.ds(c2 * CHD, CHD)], dst_c)

        def step(w, carry2):
            pltpu.sync_copy(ones_v, dacc.at[dst_c.at[w]], add=True)
            return carry2

        lax.fori_loop(0, CHD, step, 0)
        return carry

    lax.fori_loop(0, WPD // CHD, chunk, 0)

    plsc.subcore_barrier()
    pltpu.sync_copy(dacc.at[pl.ds(r0, ROWS_PER_SUB)],
                    deg.at[c].at[pl.ds(r0, ROWS_PER_SUB)])


def _make_sc():
    scratch = [
        pltpu.VMEM((CH, K), jnp.int32),
        pltpu.VMEM((CH, K), jnp.int32),
        pltpu.VMEM((K, HALF), jnp.float32),
        pltpu.VMEM((K, HALF), jnp.float32),
        pltpu.VMEM((K, HALF), jnp.float32),
        pltpu.VMEM((ZCH, HALF), jnp.float32),
        pltpu.VMEM_SHARED((NPAD, HALF), jnp.float32),
        pltpu.SemaphoreType.DMA, pltpu.SemaphoreType.DMA,
        pltpu.SemaphoreType.DMA,
    ]
    mesh = plsc.VectorSubcoreMesh(core_axis_name="c", subcore_axis_name="s",
                                  num_cores=NC, num_subcores=NS)
    return pl.kernel(
        _sc_body,
        out_type=jax.ShapeDtypeStruct((NC, NPAD, HALF), jnp.float32),
        mesh=mesh,
        scratch_types=scratch,
        compiler_params=pltpu.CompilerParams(use_tc_tiling_on_sc=False),
        name="sc_segsum",
    )


def _make_sc_deg():
    scratch = [
        pltpu.VMEM((CHD, K), jnp.int32),
        pltpu.VMEM((K, 16), jnp.float32),
        pltpu.VMEM((64, 16), jnp.float32),
        pltpu.VMEM_SHARED((NPAD, 16), jnp.float32),
    ]
    mesh = plsc.VectorSubcoreMesh(core_axis_name="c", subcore_axis_name="s",
                                  num_cores=NC, num_subcores=NS)
    return pl.kernel(
        _sc_deg_body,
        out_type=jax.ShapeDtypeStruct((NC, NPAD, 16), jnp.float32),
        mesh=mesh,
        scratch_types=scratch,
        compiler_params=pltpu.CompilerParams(use_tc_tiling_on_sc=False),
        name="sc_deg",
    )


_sc_cache = {}


def _sc_segsum(xp, src, dst):
    if "agg" not in _sc_cache:
        _sc_cache["agg"] = _make_sc()
    return _sc_cache["agg"](xp, src, dst)


def _sc_degcount(dst):
    if "deg" not in _sc_cache:
        _sc_cache["deg"] = _make_sc_deg()
    return _sc_cache["deg"](dst)


R = 1000
G = N // R


def _row(spec_shape):
    nd = len(spec_shape)
    if nd == 2:
        return pl.BlockSpec((R, spec_shape[1]), lambda i: (i, 0))
    return pl.BlockSpec((spec_shape[0], R, spec_shape[2]), lambda i: (0, i, 0))


def _full(shape):
    nd = len(shape)
    return pl.BlockSpec(shape, lambda i: (0,) * nd)


def _in_proj_body(x, w1, b1, w2, b2, lg, lb, blg, blb, wp, bp, wr,
                  xcur_o, xp_o, xr_o):
    h = _gelu(jnp.dot(x[...], w1[...], preferred_element_type=jnp.float32)
              + b1[...])
    h = jnp.dot(h, w2[...], preferred_element_type=jnp.float32) + b2[...]
    h = _ln(h, lg[...], lb[...])
    xcur_o[...] = h
    xln = _ln(h, blg[...], blb[...])
    xp = jnp.maximum(
        jnp.dot(xln, wp[...], preferred_element_type=jnp.float32) + bp[...],
        0.0)
    xp_o[0] = xp[:, :HALF]
    xp_o[1] = xp[:, HALF:]
    xr_o[...] = jnp.dot(xln, wr[...], preferred_element_type=jnp.float32)


def _in_proj(x, W1, b1, W2, b2, lg, lb, blg, blb, wp, bp, wr):
    return pl.pallas_call(
        _in_proj_body,
        grid=(G,),
        in_specs=[
            _row((N, IN_DIM)),
            _full((IN_DIM, 2 * HID)), _full((1, 2 * HID)),
            _full((2 * HID, HID)), _full((1, HID)),
            _full((1, HID)), _full((1, HID)),
            _full((1, HID)), _full((1, HID)),
            _full((HID, HID)), _full((1, HID)),
            _full((HID, HID)),
        ],
        out_specs=[_row((N, HID)), _row((2, N, HALF)), _row((N, HID))],
        out_shape=[
            jax.ShapeDtypeStruct((N, HID), jnp.float32),
            jax.ShapeDtypeStruct((NC, N, HALF), jnp.float32),
            jax.ShapeDtypeStruct((N, HID), jnp.float32),
        ],
        name="tc_in_proj",
    )(x, W1, b1, W2, b2, lg, lb, blg, blb, wp, bp, wr)


def _post_body(last, xcur, xr, agg, deg, wl, bl, wg, bg,
               blg, blb, wp, bp, wr, *outs):
    d = deg[0, :, 0:1] + deg[1, :, 0:1]
    mean = jnp.concatenate([agg[0], agg[1]], axis=-1) / jnp.maximum(d, 1.0)
    out = (jnp.dot(mean, wl[...], preferred_element_type=jnp.float32)
           + bl[...] + xr[...])
    nrm = jnp.sqrt(jnp.sum(out * out, axis=-1, keepdims=True))
    out = out / jnp.maximum(nrm, 1e-12)
    hi = xcur[...] + _gelu(out)
    xc = xcur[...]
    gate_in = jnp.concatenate([xc, hi], axis=-1)
    g = jax.nn.sigmoid(
        jnp.dot(gate_in, wg[...], preferred_element_type=jnp.float32)
        + bg[...])
    xnew = xc + g * hi
    if last:
        outs[0][...] = _ln(xnew, blg[...], blb[...])
    else:
        xcur_o, xp_o, xr_o = outs
        xcur_o[...] = xnew
        xln = _ln(xnew, blg[...], blb[...])
        xp = jnp.maximum(
            jnp.dot(xln, wp[...], preferred_element_type=jnp.float32)
            + bp[...], 0.0)
        xp_o[0] = xp[:, :HALF]
        xp_o[1] = xp[:, HALF:]
        xr_o[...] = jnp.dot(xln, wr[...], preferred_element_type=jnp.float32)


def _post(last, xcur, xr, agg, deg, wl, bl, wg, bg, blg, blb, wp, bp, wr):
    if last:
        out_specs = [_row((N, HID))]
        out_shape = [jax.ShapeDtypeStruct((N, HID), jnp.float32)]
    else:
        out_specs = [_row((N, HID)), _row((2, N, HALF)), _row((N, HID))]
        out_shape = [
            jax.ShapeDtypeStruct((N, HID), jnp.float32),
            jax.ShapeDtypeStruct((NC, N, HALF), jnp.float32),
            jax.ShapeDtypeStruct((N, HID), jnp.float32),
        ]
    res = pl.pallas_call(
        functools.partial(_post_body, last),
        grid=(G,),
        in_specs=[
            _row((N, HID)), _row((N, HID)),
            _row((2, N, HALF)),
            pl.BlockSpec((2, R, 16), lambda i: (0, i, 0)),
            _full((HID, HID)), _full((1, HID)),
            _full((2 * HID, HID)), _full((1, HID)),
            _full((1, HID)), _full((1, HID)),
            _full((HID, HID)), _full((1, HID)),
            _full((HID, HID)),
        ],
        out_specs=out_specs,
        out_shape=out_shape,
        name="tc_post_last" if last else "tc_post",
    )(xcur, xr, agg, deg, wl, bl, wg, bg, blg, blb, wp, bp, wr)
    return res


def kernel(x, edge_index, W1, b1, W2, b2, ln_in_g, ln_in_b, Wp, bp, Wl, bl,
           Wr, blk_ln_g, blk_ln_b, Wg, bg, fn_g, fn_b):
    npad_e = EPAD - E
    fill = jnp.arange(npad_e, dtype=jnp.int32)
    src = jnp.concatenate([edge_index[0], fill % N]).reshape(NS, WPT, K)
    dst = jnp.concatenate([edge_index[1], N + fill % (NPAD - N)]
                          ).reshape(NS, WPT, K)
    r2 = lambda a: a.reshape(1, -1)

    xcur, xp, xr = _in_proj(
        x, W1, r2(b1), W2, r2(b2), r2(ln_in_g), r2(ln_in_b),
        r2(blk_ln_g[0]), r2(blk_ln_b[0]), Wp[0], r2(bp[0]), Wr[0])

    deg = _sc_degcount(dst.reshape(NC * NS, WPD, K))
    for i in range(L):
        agg = _sc_segsum(xp, src, dst)
        last = i == L - 1
        if last:
            nblg, nblb = fn_g, fn_b
            nwp, nbp, nwr = Wp[0], bp[0], Wr[0]
        else:
            nblg, nblb = blk_ln_g[i + 1], blk_ln_b[i + 1]
            nwp, nbp, nwr = Wp[i + 1], bp[i + 1], Wr[i + 1]
        res = _post(last, xcur, xr, agg, deg, Wl[i], r2(bl[i]), Wg[i],
                    r2(bg[i]), r2(nblg), r2(nblb), nwp, r2(nbp), nwr)
        if last:
            return res[0]
        xcur, xp, xr = res


# speedup vs baseline: 9.8435x; 1.0211x over previous
import functools

import jax
import jax.numpy as jnp
from jax import lax
from jax.experimental import pallas as pl
from jax.experimental.pallas import tpu as pltpu
from jax.experimental.pallas import tpu_sc as plsc

N = 10000
E = 320000
IN_DIM = 128
HID = 256
L = 4

NC = 2
NS = 16
K = 96
WPT = 210
EPAD = NS * WPT * K
NPAD = 10240
ROWS_PER_SUB = NPAD // NS
HALF = HID // 2
ZCH = 16
NB = 2
NBUF = 3
CH = 30
NCH = WPT // CH
CHD = 21


def _ln(x, g, b):
    m = x.mean(-1, keepdims=True)
    v = ((x - m) ** 2).mean(-1, keepdims=True)
    return (x - m) * jax.lax.rsqrt(v + 1e-5) * g + b


def _gelu(x):
    return 0.5 * x * (1.0 + lax.erf(x * (2.0 ** -0.5)))


def _zero_shared(zb, dest, r0, nrows, width):
    zeros16 = jnp.zeros((16,), jnp.float32)
    zrows = zb.shape[0]

    def zrow(i, carry):
        for j in range(width // 16):
            zb[i, pl.ds(j * 16, 16)] = zeros16
        return carry

    lax.fori_loop(0, zrows, zrow, 0)

    def zcopy(i, carry):
        pltpu.sync_copy(zb, dest.at[pl.ds(r0 + i * zrows, zrows)])
        return carry

    lax.fori_loop(0, nrows // zrows, zcopy, 0)


def _sc_body(xp, src_h, dst_h, zq, agg, src_c, dst_c, rows0, rows1, rows2,
             zbuf, acc, sg0, sg1, sg2):
    rows = (rows0, rows1, rows2)
    sg = (sg0, sg1, sg2)

    c = lax.axis_index("c")
    s = lax.axis_index("s")
    r0 = s * ROWS_PER_SUB

    pltpu.sync_copy(zq, zbuf)

    def zcopy(i, carry):
        pltpu.sync_copy(zbuf, acc.at[pl.ds(r0 + i * ZCH, ZCH)])
        return carry

    lax.fori_loop(0, ROWS_PER_SUB // ZCH, zcopy, 0)
    plsc.subcore_barrier()

    table = xp.at[c]

    def start_gather(w, b):
        pltpu.make_async_copy(table.at[src_c.at[w]], rows[b], sg[b]).start()

    def wait_gather(w, b):
        pltpu.make_async_copy(table.at[src_c.at[w]], rows[b], sg[b]).wait()

    def scatter(w, b):
        pltpu.sync_copy(rows[b], acc.at[dst_c.at[w]], add=True)

    def chunk(c2, carry):
        pltpu.sync_copy(src_h.at[s].at[pl.ds(c2 * CH, CH)], src_c)
        pltpu.sync_copy(dst_h.at[s].at[pl.ds(c2 * CH, CH)], dst_c)
        for b in range(NB):
            start_gather(b, b)

        def step(g, carry2):
            for j in range(NBUF):
                w = g * NBUF + j
                wait_gather(w, j)
                start_gather(w + NB, (j + NB) % NBUF)
                scatter(w, j)
            return carry2

        lax.fori_loop(0, (CH - NBUF) // NBUF, step, 0)
        for w in range(CH - NBUF, CH):
            b = w % NBUF
            wait_gather(w, b)
            if w + NB < CH:
                start_gather(w + NB, (w + NB) % NBUF)
            scatter(w, b)
        return carry

    lax.fori_loop(0, NCH, chunk, 0)

    plsc.subcore_barrier()
    pltpu.sync_copy(acc.at[pl.ds(r0, ROWS_PER_SUB)],
                    agg.at[c].at[pl.ds(r0, ROWS_PER_SUB)])


WPD = EPAD // (NC * NS) // K


def _sc_deg_body(dst_h, deg, dst_c, ones_v, zdbuf, dacc):
    c = lax.axis_index("c")
    s = lax.axis_index("s")
    wid = s * NC + c
    r0 = s * ROWS_PER_SUB

    _zero_shared(zdbuf, dacc, r0, ROWS_PER_SUB, 16)

    def orow(i, carry):
        ones_v[i, pl.ds(0, 16)] = jnp.ones((16,), jnp.float32)
        return carry

    lax.fori_loop(0, K, orow, 0)
    plsc.subcore_barrier()

    def chunk(c2, carry):
        pltpu.sync_copy(dst_h.at[wid].at[pl.ds(c2 * CHD, CHD)], dst_c)

        def step(w, carry2):
            pltpu.sync_copy(ones_v, dacc.at[dst_c.at[w]], add=True)
            return carry2

        lax.fori_loop(0, CHD, step, 0)
        return carry

    lax.fori_loop(0, WPD // CHD, chunk, 0)

    plsc.subcore_barrier()
    pltpu.sync_copy(dacc.at[pl.ds(r0, ROWS_PER_SUB)],
                    deg.at[c].at[pl.ds(r0, ROWS_PER_SUB)])


def _make_sc():
    scratch = [
        pltpu.VMEM((CH, K), jnp.int32),
        pltpu.VMEM((CH, K), jnp.int32),
        pltpu.VMEM((K, HALF), jnp.int16),
        pltpu.VMEM((K, HALF), jnp.int16),
        pltpu.VMEM((K, HALF), jnp.int16),
        pltpu.VMEM((ZCH, HALF), jnp.int16),
        pltpu.VMEM_SHARED((NPAD, HALF), jnp.int16),
        pltpu.SemaphoreType.DMA, pltpu.SemaphoreType.DMA,
        pltpu.SemaphoreType.DMA,
    ]
    mesh = plsc.VectorSubcoreMesh(core_axis_name="c", subcore_axis_name="s",
                                  num_cores=NC, num_subcores=NS)
    return pl.kernel(
        _sc_body,
        out_type=jax.ShapeDtypeStruct((NC, NPAD, HALF), jnp.int16),
        mesh=mesh,
        scratch_types=scratch,
        compiler_params=pltpu.CompilerParams(use_tc_tiling_on_sc=False),
        name="sc_segsum",
    )


def _make_sc_deg():
    scratch = [
        pltpu.VMEM((CHD, K), jnp.int32),
        pltpu.VMEM((K, 16), jnp.float32),
        pltpu.VMEM((64, 16), jnp.float32),
        pltpu.VMEM_SHARED((NPAD, 16), jnp.float32),
    ]
    mesh = plsc.VectorSubcoreMesh(core_axis_name="c", subcore_axis_name="s",
                                  num_cores=NC, num_subcores=NS)
    return pl.kernel(
        _sc_deg_body,
        out_type=jax.ShapeDtypeStruct((NC, NPAD, 16), jnp.float32),
        mesh=mesh,
        scratch_types=scratch,
        compiler_params=pltpu.CompilerParams(use_tc_tiling_on_sc=False),
        name="sc_deg",
    )


_sc_cache = {}


def _sc_segsum(xp, src, dst, zq):
    if "agg" not in _sc_cache:
        _sc_cache["agg"] = _make_sc()
    return _sc_cache["agg"](xp, src, dst, zq)


def _sc_degcount(dst):
    if "deg" not in _sc_cache:
        _sc_cache["deg"] = _make_sc_deg()
    return _sc_cache["deg"](dst)


R = 1000
G = N // R


def _row(spec_shape):
    nd = len(spec_shape)
    if nd == 2:
        return pl.BlockSpec((R, spec_shape[1]), lambda i: (i, 0))
    return pl.BlockSpec((spec_shape[0], R, spec_shape[2]), lambda i: (0, i, 0))


def _full(shape):
    nd = len(shape)
    return pl.BlockSpec(shape, lambda i: (0,) * nd)


def _in_proj_body(x, w1, b1, w2, b2, lg, lb, blg, blb, wp, bp, wr,
                  xcur_o, xp_o, xr_o, xmax_o):
    h = _gelu(jnp.dot(x[...], w1[...], preferred_element_type=jnp.float32)
              + b1[...])
    h = jnp.dot(h, w2[...], preferred_element_type=jnp.float32) + b2[...]
    h = _ln(h, lg[...], lb[...])
    xcur_o[...] = h
    xln = _ln(h, blg[...], blb[...])
    xp = jnp.maximum(
        jnp.dot(xln, wp[...], preferred_element_type=jnp.float32) + bp[...],
        0.0)
    xp_o[0] = xp[:, :HALF]
    xp_o[1] = xp[:, HALF:]
    xr_o[...] = jnp.dot(xln, wr[...], preferred_element_type=jnp.float32)
    xmax_o[...] = jnp.broadcast_to(jnp.max(xp).reshape(1, 1), (8, 128))


def _in_proj(x, W1, b1, W2, b2, lg, lb, blg, blb, wp, bp, wr):
    return pl.pallas_call(
        _in_proj_body,
        grid=(G,),
        in_specs=[
            _row((N, IN_DIM)),
            _full((IN_DIM, 2 * HID)), _full((1, 2 * HID)),
            _full((2 * HID, HID)), _full((1, HID)),
            _full((1, HID)), _full((1, HID)),
            _full((1, HID)), _full((1, HID)),
            _full((HID, HID)), _full((1, HID)),
            _full((HID, HID)),
        ],
        out_specs=[_row((N, HID)), _row((2, N, HALF)), _row((N, HID)),
                   pl.BlockSpec((8, 128), lambda i: (0, i))],
        out_shape=[
            jax.ShapeDtypeStruct((N, HID), jnp.float32),
            jax.ShapeDtypeStruct((NC, N, HALF), jnp.float32),
            jax.ShapeDtypeStruct((N, HID), jnp.float32),
            jax.ShapeDtypeStruct((8, G * 128), jnp.float32),
        ],
        name="tc_in_proj",
    )(x, W1, b1, W2, b2, lg, lb, blg, blb, wp, bp, wr)


def _quant_body(xp, s, xq_o):
    xq_o[...] = jnp.round(xp[...] * s[0, 0]).astype(jnp.int16)


def _quant(xp, s):
    return pl.pallas_call(
        _quant_body,
        grid=(G,),
        in_specs=[_row((2, N, HALF)), _full((1, 1))],
        out_specs=_row((2, N, HALF)),
        out_shape=jax.ShapeDtypeStruct((NC, N, HALF), jnp.int16),
        name="tc_quant",
    )(xp, s)


def _post_body(last, xcur, xr, agg, deg, isc, wl, bl, wg, bg,
               blg, blb, wp, bp, wr, *outs):
    d = deg[0, :, 0:1] + deg[1, :, 0:1]
    aggf = jnp.concatenate([agg[0], agg[1]], axis=-1).astype(jnp.float32)
    mean = aggf * (isc[0, 0] / jnp.maximum(d, 1.0))
    out = (jnp.dot(mean, wl[...], preferred_element_type=jnp.float32)
           + bl[...] + xr[...])
    nrm = jnp.sqrt(jnp.sum(out * out, axis=-1, keepdims=True))
    out = out / jnp.maximum(nrm, 1e-12)
    hi = xcur[...] + _gelu(out)
    xc = xcur[...]
    gate_in = jnp.concatenate([xc, hi], axis=-1)
    g = jax.nn.sigmoid(
        jnp.dot(gate_in, wg[...], preferred_element_type=jnp.float32)
        + bg[...])
    xnew = xc + g * hi
    if last:
        outs[0][...] = _ln(xnew, blg[...], blb[...])
    else:
        xcur_o, xp_o, xr_o, xmax_o = outs
        xcur_o[...] = xnew
        xln = _ln(xnew, blg[...], blb[...])
        xp = jnp.maximum(
            jnp.dot(xln, wp[...], preferred_element_type=jnp.float32)
            + bp[...], 0.0)
        xp_o[0] = xp[:, :HALF]
        xp_o[1] = xp[:, HALF:]
        xr_o[...] = jnp.dot(xln, wr[...], preferred_element_type=jnp.float32)
        xmax_o[...] = jnp.broadcast_to(jnp.max(xp).reshape(1, 1), (8, 128))


def _post(last, xcur, xr, agg, deg, isc, wl, bl, wg, bg, blg, blb, wp, bp,
          wr):
    if last:
        out_specs = [_row((N, HID))]
        out_shape = [jax.ShapeDtypeStruct((N, HID), jnp.float32)]
    else:
        out_specs = [_row((N, HID)), _row((2, N, HALF)), _row((N, HID)),
                     pl.BlockSpec((8, 128), lambda i: (0, i))]
        out_shape = [
            jax.ShapeDtypeStruct((N, HID), jnp.float32),
            jax.ShapeDtypeStruct((NC, N, HALF), jnp.float32),
            jax.ShapeDtypeStruct((N, HID), jnp.float32),
            jax.ShapeDtypeStruct((8, G * 128), jnp.float32),
        ]
    res = pl.pallas_call(
        functools.partial(_post_body, last),
        grid=(G,),
        in_specs=[
            _row((N, HID)), _row((N, HID)),
            _row((2, N, HALF)),
            pl.BlockSpec((2, R, 16), lambda i: (0, i, 0)),
            _full((1, 1)),
            _full((HID, HID)), _full((1, HID)),
            _full((2 * HID, HID)), _full((1, HID)),
            _full((1, HID)), _full((1, HID)),
            _full((HID, HID)), _full((1, HID)),
            _full((HID, HID)),
        ],
        out_specs=out_specs,
        out_shape=out_shape,
        name="tc_post_last" if last else "tc_post",
    )(xcur, xr, agg, deg, isc, wl, bl, wg, bg, blg, blb, wp, bp, wr)
    return res


def kernel(x, edge_index, W1, b1, W2, b2, ln_in_g, ln_in_b, Wp, bp, Wl, bl,
           Wr, blk_ln_g, blk_ln_b, Wg, bg, fn_g, fn_b):
    npad_e = EPAD - E
    fill = jnp.arange(npad_e, dtype=jnp.int32)
    src = jnp.concatenate([edge_index[0], fill % N]).reshape(NS, WPT, K)
    dst = jnp.concatenate([edge_index[1], N + fill % (NPAD - N)]
                          ).reshape(NS, WPT, K)
    r2 = lambda a: a.reshape(1, -1)

    xcur, xp, xr, xmax = _in_proj(
        x, W1, r2(b1), W2, r2(b2), r2(ln_in_g), r2(ln_in_b),
        r2(blk_ln_g[0]), r2(blk_ln_b[0]), Wp[0], r2(bp[0]), Wr[0])

    deg = _sc_degcount(dst.reshape(NC * NS, WPD, K))
    degmax = jnp.maximum(jnp.max(deg[0, :N, 0] + deg[1, :N, 0]), 1.0)
    zq = jnp.zeros((ZCH, HALF), jnp.int16)
    for i in range(L):
        scale = 30000.0 / (jnp.maximum(jnp.max(xmax), 1e-30) * degmax)
        xq = _quant(xp, scale.reshape(1, 1))
        agg = _sc_segsum(xq, src, dst, zq)
        isc = (1.0 / scale).reshape(1, 1)
        last = i == L - 1
        if last:
            nblg, nblb = fn_g, fn_b
            nwp, nbp, nwr = Wp[0], bp[0], Wr[0]
        else:
            nblg, nblb = blk_ln_g[i + 1], blk_ln_b[i + 1]
            nwp, nbp, nwr = Wp[i + 1], bp[i + 1], Wr[i + 1]
        res = _post(last, xcur, xr, agg, deg, isc, Wl[i], r2(bl[i]), Wg[i],
                    r2(bg[i]), r2(nblg), r2(nblb), nwp, r2(nbp), nwr)
        if last:
            return res[0]
        xcur, xp, xr, xmax = res
